# baseline (speedup 1.0000x reference)
import jax
import jax.numpy as jnp
from jax import lax
from jax.experimental import pallas as pl
from jax.experimental.pallas import tpu as pltpu
from jax.experimental.pallas import tpu_sc as plsc

_L = 16
_B = 64
_RPT = 8
_NTILES = _B // _RPT
_W = 128

_DN = lax.GatherDimensionNumbers(
    offset_dims=(), collapsed_slice_dims=(0,), start_index_map=(0,))


def _sc_body(table_hbm, rpi_hbm, plen_hbm, out_hbm,
             rpi_v, plen_v, chunks_v, out_v, sem, sem_in):
    wid = lax.axis_index("s")

    @pl.when(wid < _NTILES)
    def _():
        base = wid * _RPT
        cp_r = pltpu.async_copy(rpi_hbm.at[pl.ds(base, _RPT)],
                                rpi_v.at[pl.ds(0, _RPT)], sem_in)
        cp_p = pltpu.async_copy(plen_hbm.at[pl.ds(base, _RPT)],
                                plen_v.at[pl.ds(0, _RPT)], sem_in)
        cp_r.wait()
        cp_p.wait()
        r = rpi_v[...]
        p = plen_v[...]
        c = jnp.maximum(p - 1, 0)
        copies = []
        for i in range(_RPT):
            ri = r[i]
            ci = c[i]
            r0i = pl.multiple_of((ri >> 3) << 3, 8)
            c0i = pl.multiple_of((ci >> 7) << 7, _W)
            di = 16 * i + 7 - (ri & 7)
            copies.append(
                pltpu.async_copy(table_hbm.at[pl.ds(r0i, 8), pl.ds(c0i, _W)],
                                 chunks_v.at[pl.ds(di, 8)], sem))
        for cp in copies:
            cp.wait()
        lane = lax.iota(jnp.int32, _L)
        acc = jnp.full((_L,), -1, jnp.int32)
        for i in range(_RPT):
            ci = c[i]
            cw = pl.multiple_of(((ci & (_W - 1)) >> 4) << 4, 16)
            w = chunks_v[16 * i + 7, pl.ds(cw, 16)]
            idx = jnp.full((_L, 1), ci & 15, jnp.int32)
            g = lax.gather(w, idx, _DN, (1,),
                           mode=lax.GatherScatterMode.PROMISE_IN_BOUNDS)
            acc = jnp.where(lane == i, g, acc)
        out_v[...] = jnp.where(p > 0, acc, jnp.full_like(p, -1))
        pltpu.sync_copy(out_v.at[pl.ds(0, _RPT)], out_hbm.at[pl.ds(base, _RPT)])


def kernel(req_to_token, req_pool_indices_tensor, prefix_lens_tensor):
    out_dtype = prefix_lens_tensor.dtype
    table = req_to_token.astype(jnp.int32)
    rpi = req_pool_indices_tensor.astype(jnp.int32)
    plen = prefix_lens_tensor.astype(jnp.int32)

    mesh = plsc.VectorSubcoreMesh(core_axis_name="c", subcore_axis_name="s",
                                  num_cores=1)
    f = pl.kernel(
        _sc_body,
        out_type=jax.ShapeDtypeStruct((_B,), jnp.int32),
        mesh=mesh,
        scratch_types=[
            pltpu.VMEM((_L,), jnp.int32),
            pltpu.VMEM((_L,), jnp.int32),
            pltpu.VMEM((16 * _RPT, _W), jnp.int32),
            pltpu.VMEM((_L,), jnp.int32),
            pltpu.SemaphoreType.DMA,
            pltpu.SemaphoreType.DMA,
        ],
    )
    out = f(table, rpi, plen)
    return out.astype(out_dtype)


# speedup vs baseline: 1.0031x; 1.0031x over previous
import jax
import jax.numpy as jnp
from jax import lax
from jax.experimental import pallas as pl
from jax.experimental.pallas import tpu as pltpu
from jax.experimental.pallas import tpu_sc as plsc

_L = 16
_B = 64
_RPT = 8
_NTILES = _B // _RPT
_W = 128

_DN = lax.GatherDimensionNumbers(
    offset_dims=(), collapsed_slice_dims=(0,), start_index_map=(0,))


def _sc_body(table_hbm, rpi_hbm, plen_hbm, out_hbm,
             rpi_v, plen_v, chunks_v, out_v, sem, sem_in):
    wid = lax.axis_index("s")

    @pl.when(wid < _NTILES)
    def _():
        base = wid * _RPT
        cp_r = pltpu.async_copy(rpi_hbm.at[pl.ds(base, _RPT)],
                                rpi_v.at[pl.ds(0, _RPT)], sem_in)
        cp_p = pltpu.async_copy(plen_hbm.at[pl.ds(base, _RPT)],
                                plen_v.at[pl.ds(0, _RPT)], sem_in)
        cp_r.wait()
        cp_p.wait()
        r = rpi_v[...]
        p = plen_v[...]
        c = jnp.maximum(p - 1, 0)
        half = _RPT // 2
        sems = [sem, sem_in]
        copies = []
        for i in range(_RPT):
            ri = r[i]
            ci = c[i]
            r0i = pl.multiple_of((ri >> 3) << 3, 8)
            c0i = pl.multiple_of((ci >> 7) << 7, _W)
            di = 16 * i + 7 - (ri & 7)
            copies.append(
                pltpu.async_copy(table_hbm.at[pl.ds(r0i, 8), pl.ds(c0i, _W)],
                                 chunks_v.at[pl.ds(di, 8)], sems[i // half]))
        lane = lax.iota(jnp.int32, _L)
        acc = jnp.full((_L,), -1, jnp.int32)
        for i in range(_RPT):
            if i % half == 0:
                for cp in copies[i:i + half]:
                    cp.wait()
            ci = c[i]
            cw = pl.multiple_of(((ci & (_W - 1)) >> 4) << 4, 16)
            w = chunks_v[16 * i + 7, pl.ds(cw, 16)]
            idx = jnp.full((_L, 1), ci & 15, jnp.int32)
            g = lax.gather(w, idx, _DN, (1,),
                           mode=lax.GatherScatterMode.PROMISE_IN_BOUNDS)
            acc = jnp.where(lane == i, g, acc)
        out_v[...] = jnp.where(p > 0, acc, jnp.full_like(p, -1))
        pltpu.sync_copy(out_v.at[pl.ds(0, _RPT)], out_hbm.at[pl.ds(base, _RPT)])


def kernel(req_to_token, req_pool_indices_tensor, prefix_lens_tensor):
    out_dtype = prefix_lens_tensor.dtype
    table = req_to_token.astype(jnp.int32)
    rpi = req_pool_indices_tensor.astype(jnp.int32)
    plen = prefix_lens_tensor.astype(jnp.int32)

    mesh = plsc.VectorSubcoreMesh(core_axis_name="c", subcore_axis_name="s",
                                  num_cores=1)
    f = pl.kernel(
        _sc_body,
        out_type=jax.ShapeDtypeStruct((_B,), jnp.int32),
        mesh=mesh,
        scratch_types=[
            pltpu.VMEM((_L,), jnp.int32),
            pltpu.VMEM((_L,), jnp.int32),
            pltpu.VMEM((16 * _RPT, _W), jnp.int32),
            pltpu.VMEM((_L,), jnp.int32),
            pltpu.SemaphoreType.DMA,
            pltpu.SemaphoreType.DMA,
        ],
    )
    out = f(table, rpi, plen)
    return out.astype(out_dtype)
